# baseline (speedup 1.0000x reference)
import functools

import jax
import jax.numpy as jnp
import numpy as np
from jax.experimental import pallas as pl
from jax.experimental.pallas import tpu as pltpu

_SA_CFGS = [
    {'npoint': 1024, 'radius': 0.1, 'nsample': 32},
    {'npoint': 256, 'radius': 0.2, 'nsample': 32},
    {'npoint': 64, 'radius': 0.4, 'nsample': 32},
    {'npoint': 16, 'radius': 0.8, 'nsample': 32},
]
_ALPHA = 0.2
_BN_SCALE = 1.0 / np.sqrt(1.0 + 1e-5)


def _fps_body(npoint, N, xyz_ref, idx_ref):
    xs = xyz_ref[:, 0, :]
    ys = xyz_ref[:, 1, :]
    zs = xyz_ref[:, 2, :]
    iota = jax.lax.broadcasted_iota(jnp.int32, (1, N), 1)
    iota_np = jax.lax.broadcasted_iota(jnp.int32, (1, npoint), 1)

    idx_ref[:, 0, :] = jnp.zeros((1, npoint), jnp.int32)

    def body(i, st):
        dist, far = st
        idx_ref[:, 0, :] = jnp.where(iota_np == i, far, idx_ref[:, 0, :])
        oh = (iota == far).astype(jnp.float32)
        cx = jnp.sum(xs * oh, 1, keepdims=True)
        cy = jnp.sum(ys * oh, 1, keepdims=True)
        cz = jnp.sum(zs * oh, 1, keepdims=True)
        dx = xs - cx
        dy = ys - cy
        dz = zs - cz
        d = dx * dx + dy * dy + dz * dz
        dist = jnp.minimum(dist, d)
        m = jnp.max(dist, 1, keepdims=True)
        far = jnp.min(jnp.where(dist == m, iota, N), 1, keepdims=True)
        return dist, far

    dist0 = jnp.full((1, N), 1e10, jnp.float32)
    far0 = jnp.zeros((1, 1), jnp.int32)
    jax.lax.fori_loop(0, npoint, body, (dist0, far0))


def _fps_call(xyz_c, npoint):
    B, _, N = xyz_c.shape
    out = pl.pallas_call(
        functools.partial(_fps_body, npoint, N),
        grid=(B,),
        in_specs=[pl.BlockSpec((1, 3, N), lambda b: (b, 0, 0))],
        out_specs=pl.BlockSpec((1, 1, npoint), lambda b: (b, 0, 0)),
        out_shape=jax.ShapeDtypeStruct((B, 1, npoint), jnp.int32),
        compiler_params=pltpu.CompilerParams(
            dimension_semantics=("parallel",)),
    )(xyz_c)
    return out


def _group_body(N, C3, nsample, r2, fidx_ref, table_ref, fps_ref, grp_ref):
    table = table_ref[0]
    fidx = fidx_ref[0, 0, :]
    chunk = fidx.shape[0]
    col = jax.lax.broadcasted_iota(jnp.int32, (chunk, N), 1)

    foh = (col == fidx[:, None]).astype(jnp.float32)
    fps_pts = jnp.dot(foh, table, preferred_element_type=jnp.float32, precision=jax.lax.Precision.HIGHEST)
    fps_ref[0] = fps_pts
    src3 = fps_pts[:, :3]
    t3 = table[:, :3]
    sqr = -2.0 * jax.lax.dot_general(
        src3, t3, (((1,), (1,)), ((), ())), preferred_element_type=jnp.float32)
    sqr = sqr + jnp.sum(src3 * src3, 1, keepdims=True)
    sqr = sqr + jnp.sum(t3 * t3, axis=1)[None, :]
    avail = sqr <= r2

    idx0 = None
    for k in range(nsample):
        cand = jnp.where(avail, col, N)
        ik = jnp.min(cand, 1, keepdims=True)
        if k == 0:
            idx0 = jnp.minimum(ik, N - 1)
            iku = idx0
        else:
            iku = jnp.where(ik < N, ik, idx0)
        avail = jnp.logical_and(avail, col != ik)
        oh = (col == iku).astype(jnp.float32)
        row = jnp.dot(oh, table, preferred_element_type=jnp.float32, precision=jax.lax.Precision.HIGHEST)
        gxn = row[:, :3] - src3
        grp_ref[0, k] = jnp.concatenate([gxn, row[:, 3:]], 1)


def _group_call(table, fidx, r2, nsample, chunk):
    B, N, C3 = table.shape
    S = fidx.shape[2]
    grid = (B, S // chunk)
    fps_pts, grp = pl.pallas_call(
        functools.partial(_group_body, N, C3, nsample, r2),
        grid=grid,
        in_specs=[
            pl.BlockSpec((1, 1, chunk), lambda b, s: (b, 0, s)),
            pl.BlockSpec((1, N, C3), lambda b, s: (b, 0, 0)),
        ],
        out_specs=[
            pl.BlockSpec((1, chunk, C3), lambda b, s: (b, s, 0)),
            pl.BlockSpec((1, nsample, chunk, C3), lambda b, s: (b, 0, s, 0)),
        ],
        compiler_params=pltpu.CompilerParams(
            dimension_semantics=("parallel", "parallel")),
        out_shape=[
            jax.ShapeDtypeStruct((B, S, C3), jnp.float32),
            jax.ShapeDtypeStruct((B, nsample, S, C3), jnp.float32),
        ],
    )(fidx, table)
    return fps_pts, grp


def _mlp_chain(h, wrefs):
    for (w_ref, b_ref, s_ref, t_ref) in wrefs:
        h = jnp.dot(h, w_ref[...], preferred_element_type=jnp.float32)
        h = (h + b_ref[...]) * s_ref[...] + t_ref[...]
        h = jnp.maximum(h, 0.0)
    return h


def _attn_body(nsample, nlayers, *refs):
    grp_ref, fps_ref = refs[0], refs[1]
    wrefs = [tuple(refs[2 + 4 * i: 6 + 4 * i]) for i in range(nlayers)]
    ap_ref, ah_ref = refs[2 + 4 * nlayers], refs[3 + 4 * nlayers]
    out_ref = refs[4 + 4 * nlayers]

    g4 = grp_ref[0]
    ns, chunk, cin = g4.shape
    g = g4.reshape(ns * chunk, cin)
    gxn = g[:, :3]
    h = _mlp_chain(g, wrefs)
    hc = _mlp_chain(fps_ref[0], wrefs)
    cout = h.shape[1]

    cterm = jnp.dot(hc, ah_ref[...], preferred_element_type=jnp.float32)
    gsum = jnp.dot(gxn, ap_ref[...], preferred_element_type=jnp.float32)
    gsum = gsum + jnp.dot(h, ah_ref[...], preferred_element_type=jnp.float32)
    pre = cterm[None, :, :] - gsum.reshape(ns, chunk, cout)
    e = jnp.where(pre >= 0, pre, _ALPHA * pre)
    m = jnp.max(e, axis=0)
    ex = jnp.exp(e - m[None, :, :])
    att = ex / jnp.sum(ex, axis=0)[None, :, :]
    out_ref[0] = jnp.sum(att * h.reshape(ns, chunk, cout), axis=0)


def _attn_call(grp, fps_pts, layers, a_p, a_h, chunk):
    B, nsample, S, Cin = grp.shape
    Cout = a_h.shape[1]
    nlayers = len(layers)
    grid = (B, S // chunk)
    in_specs = [
        pl.BlockSpec((1, nsample, chunk, Cin), lambda b, s: (b, 0, s, 0)),
        pl.BlockSpec((1, chunk, Cin), lambda b, s: (b, s, 0)),
    ]
    args = [grp, fps_pts]
    for (wt, bb, sg, bt) in layers:
        for arr in (wt, bb, sg, bt):
            in_specs.append(pl.BlockSpec(arr.shape, lambda b, s: (0, 0)))
            args.append(arr)
    for arr in (a_p, a_h):
        in_specs.append(pl.BlockSpec(arr.shape, lambda b, s: (0, 0)))
        args.append(arr)
    out = pl.pallas_call(
        functools.partial(_attn_body, nsample, nlayers),
        grid=grid,
        in_specs=in_specs,
        out_specs=pl.BlockSpec((1, chunk, Cout), lambda b, s: (b, s, 0)),
        compiler_params=pltpu.CompilerParams(
            dimension_semantics=("parallel", "parallel")),
        out_shape=jax.ShapeDtypeStruct((B, S, Cout), jnp.float32),
    )(*args)
    return out


def _fp_body(nlayers, has_p1, head, *refs):
    i = 0
    x1_ref = refs[i]; i += 1
    x2_ref = refs[i]; i += 1
    p2_ref = refs[i]; i += 1
    p1_ref = None
    if has_p1:
        p1_ref = refs[i]; i += 1
    wrefs = [tuple(refs[i + 4 * j: i + 4 * j + 4]) for j in range(nlayers)]
    i += 4 * nlayers
    hrefs = None
    if head:
        hrefs = refs[i:i + 8]
        i += 8
    out_ref = refs[i]

    src = x1_ref[0]
    dst = x2_ref[0]
    p2 = p2_ref[0]
    chunk = src.shape[0]
    n2 = dst.shape[0]
    sqr = -2.0 * jax.lax.dot_general(
        src, dst, (((1,), (1,)), ((), ())), preferred_element_type=jnp.float32)
    sqr = sqr + jnp.sum(src * src, 1, keepdims=True)
    sqr = sqr + jnp.sum(dst * dst, axis=1)[None, :]
    col = jax.lax.broadcasted_iota(jnp.int32, (chunk, n2), 1)

    d = sqr
    ws = []
    rows = []
    for _ in range(3):
        mj = jnp.min(d, 1, keepdims=True)
        ij = jnp.min(jnp.where(d == mj, col, n2), 1, keepdims=True)
        oh = (col == ij).astype(jnp.float32)
        rows.append(jnp.dot(oh, p2, preferred_element_type=jnp.float32, precision=jax.lax.Precision.HIGHEST))
        ws.append(1.0 / (mj + 1e-8))
        d = jnp.where(col == ij, jnp.float32(np.inf), d)
    wsum = (ws[0] + ws[1]) + ws[2]
    interp = (ws[0] / wsum) * rows[0] + (ws[1] / wsum) * rows[1] \
        + (ws[2] / wsum) * rows[2]
    if has_p1:
        h = jnp.concatenate([p1_ref[0], interp], 1)
    else:
        h = interp
    h = _mlp_chain(h, wrefs)
    if head:
        c1w, c1b, c1s, c1t, c2w, c2b = hrefs[0], hrefs[1], hrefs[2], hrefs[3], hrefs[4], hrefs[5]
        h = jnp.dot(h, c1w[...], preferred_element_type=jnp.float32)
        h = (h + c1b[...]) * c1s[...] + c1t[...]
        h = jnp.maximum(h, 0.0)
        logits = jnp.dot(h, c2w[...], preferred_element_type=jnp.float32) + c2b[...]
        m = jnp.max(logits, 1, keepdims=True)
        sh = logits - m
        h = sh - jnp.log(jnp.sum(jnp.exp(sh), 1, keepdims=True))
    out_ref[0] = h


def _fp_call(x1r, x2r, p2, p1, layers, head_ws, chunk, cout):
    B, n1, _ = x1r.shape
    n2 = x2r.shape[1]
    grid = (B, n1 // chunk)
    in_specs = [
        pl.BlockSpec((1, chunk, 3), lambda b, s: (b, s, 0)),
        pl.BlockSpec((1, n2, 3), lambda b, s: (b, 0, 0)),
        pl.BlockSpec((1, n2, p2.shape[2]), lambda b, s: (b, 0, 0)),
    ]
    args = [x1r, x2r, p2]
    if p1 is not None:
        in_specs.append(pl.BlockSpec((1, chunk, p1.shape[2]), lambda b, s: (b, s, 0)))
        args.append(p1)
    for (wt, bb, sg, bt) in layers:
        for arr in (wt, bb, sg, bt):
            in_specs.append(pl.BlockSpec(arr.shape, lambda b, s: (0, 0)))
            args.append(arr)
    if head_ws is not None:
        for arr in head_ws:
            in_specs.append(pl.BlockSpec(arr.shape, lambda b, s: (0, 0)))
            args.append(arr)
    out = pl.pallas_call(
        functools.partial(_fp_body, len(layers), p1 is not None, head_ws is not None),
        grid=grid,
        in_specs=in_specs,
        out_specs=pl.BlockSpec((1, chunk, cout), lambda b, s: (b, s, 0)),
        compiler_params=pltpu.CompilerParams(
            dimension_semantics=("parallel", "parallel")),
        out_shape=jax.ShapeDtypeStruct((B, n1, cout), jnp.float32),
    )(*args)
    return out


def _prep_layers(mlp_params):
    out = []
    for l in mlp_params:
        wt = jnp.transpose(l['W'])
        bb = l['b'][None, :]
        sg = (_BN_SCALE * l['gamma'])[None, :]
        bt = l['beta'][None, :]
        out.append((wt, bb, sg, bt))
    return out


def kernel(xyz, points, params):
    B, _, N = xyz.shape
    xyz_c = xyz
    feats_r = jnp.transpose(points, (0, 2, 1))

    sa_chunks = [256, 256, 64, 16]
    l_xyz_c = [xyz_c]
    l_xyz_r = [jnp.transpose(xyz_c, (0, 2, 1))]
    l_feats = [feats_r]
    for li, cfg in enumerate(_SA_CFGS):
        p = params['sa%d' % (li + 1)]
        table = jnp.concatenate([l_xyz_r[-1], l_feats[-1]], -1)
        fidx = _fps_call(l_xyz_c[-1], cfg['npoint'])
        fps_pts, grp = _group_call(
            table, fidx, cfg['radius'] ** 2, cfg['nsample'], sa_chunks[li])
        layers = _prep_layers(p['mlp'])
        a_p = p['a'][:3, :]
        a_h = p['a'][3:, :]
        feats = _attn_call(grp, fps_pts, layers, a_p, a_h, sa_chunks[li])
        new_xyz_r = fps_pts[:, :, :3]
        l_xyz_c.append(jnp.transpose(new_xyz_r, (0, 2, 1)))
        l_xyz_r.append(new_xyz_r)
        l_feats.append(feats)

    fp_chunks = [64, 256, 256, 512]
    h = _fp_call(l_xyz_r[3], l_xyz_r[4], l_feats[4], l_feats[3],
                 _prep_layers(params['fp4']['mlp']), None, fp_chunks[0], 256)
    h = _fp_call(l_xyz_r[2], l_xyz_r[3], h, l_feats[2],
                 _prep_layers(params['fp3']['mlp']), None, fp_chunks[1], 256)
    h = _fp_call(l_xyz_r[1], l_xyz_r[2], h, l_feats[1],
                 _prep_layers(params['fp2']['mlp']), None, fp_chunks[2], 128)
    c1 = params['head']['c1']
    c2 = params['head']['c2']
    head_ws = (jnp.transpose(c1['W']), c1['b'][None, :],
               (_BN_SCALE * c1['gamma'])[None, :], c1['beta'][None, :],
               jnp.transpose(c2['W']), c2['b'][None, :],
               jnp.zeros((1, 1), jnp.float32), jnp.zeros((1, 1), jnp.float32))
    out = _fp_call(l_xyz_r[0], l_xyz_r[1], h, None,
                   _prep_layers(params['fp1']['mlp']), head_ws, fp_chunks[3], 13)
    return out


# speedup vs baseline: 1.4133x; 1.4133x over previous
import functools

import jax
import jax.numpy as jnp
import numpy as np
from jax.experimental import pallas as pl
from jax.experimental.pallas import tpu as pltpu

_SA_CFGS = [
    {'npoint': 1024, 'radius': 0.1, 'nsample': 32},
    {'npoint': 256, 'radius': 0.2, 'nsample': 32},
    {'npoint': 64, 'radius': 0.4, 'nsample': 32},
    {'npoint': 16, 'radius': 0.8, 'nsample': 32},
]
_ALPHA = 0.2
_BN_SCALE = 1.0 / np.sqrt(1.0 + 1e-5)


def _fps_body(npoint, N, xyz_ref, idx_ref):
    xs = xyz_ref[:, 0, :]
    ys = xyz_ref[:, 1, :]
    zs = xyz_ref[:, 2, :]
    B = xs.shape[0]
    iota = jax.lax.broadcasted_iota(jnp.int32, (B, N), 1)
    iota_np = jax.lax.broadcasted_iota(jnp.int32, (B, npoint), 1)

    idx_ref[:, 0, :] = jnp.zeros((B, npoint), jnp.int32)

    def body(i, st):
        dist, far = st
        idx_ref[:, 0, :] = jnp.where(iota_np == i, far, idx_ref[:, 0, :])
        oh = (iota == far).astype(jnp.float32)
        cx = jnp.sum(xs * oh, 1, keepdims=True)
        cy = jnp.sum(ys * oh, 1, keepdims=True)
        cz = jnp.sum(zs * oh, 1, keepdims=True)
        dx = xs - cx
        dy = ys - cy
        dz = zs - cz
        d = dx * dx + dy * dy + dz * dz
        dist = jnp.minimum(dist, d)
        m = jnp.max(dist, 1, keepdims=True)
        far = jnp.min(jnp.where(dist == m, iota, N), 1, keepdims=True)
        return dist, far

    dist0 = jnp.full((B, N), 1e10, jnp.float32)
    far0 = jnp.zeros((B, 1), jnp.int32)
    jax.lax.fori_loop(0, npoint, body, (dist0, far0))


def _fps_call(xyz_c, npoint):
    B, _, N = xyz_c.shape
    out = pl.pallas_call(
        functools.partial(_fps_body, npoint, N),
        out_shape=jax.ShapeDtypeStruct((B, 1, npoint), jnp.int32),
    )(xyz_c)
    return out


def _group_body(N, C3, nsample, r2, fidx_ref, table_ref, fps_ref, grp_ref):
    table = table_ref[0]
    fidx = fidx_ref[0, 0, :]
    chunk = fidx.shape[0]
    col = jax.lax.broadcasted_iota(jnp.int32, (chunk, N), 1)

    foh = (col == fidx[:, None]).astype(jnp.float32)
    fps_pts = jnp.dot(foh, table, preferred_element_type=jnp.float32, precision=jax.lax.Precision.HIGHEST)
    fps_ref[0] = fps_pts
    src3 = fps_pts[:, :3]
    t3 = table[:, :3]
    sqr = -2.0 * jax.lax.dot_general(
        src3, t3, (((1,), (1,)), ((), ())), preferred_element_type=jnp.float32)
    sqr = sqr + jnp.sum(src3 * src3, 1, keepdims=True)
    sqr = sqr + jnp.sum(t3 * t3, axis=1)[None, :]
    avail = sqr <= r2

    idx0 = None
    for k in range(nsample):
        cand = jnp.where(avail, col, N)
        ik = jnp.min(cand, 1, keepdims=True)
        if k == 0:
            idx0 = jnp.minimum(ik, N - 1)
            iku = idx0
        else:
            iku = jnp.where(ik < N, ik, idx0)
        avail = jnp.logical_and(avail, col != ik)
        oh = (col == iku).astype(jnp.float32)
        row = jnp.dot(oh, table, preferred_element_type=jnp.float32, precision=jax.lax.Precision.HIGHEST)
        gxn = row[:, :3] - src3
        grp_ref[0, k] = jnp.concatenate([gxn, row[:, 3:]], 1)


def _group_call(table, fidx, r2, nsample, chunk):
    B, N, C3 = table.shape
    S = fidx.shape[2]
    grid = (B, S // chunk)
    fps_pts, grp = pl.pallas_call(
        functools.partial(_group_body, N, C3, nsample, r2),
        grid=grid,
        in_specs=[
            pl.BlockSpec((1, 1, chunk), lambda b, s: (b, 0, s)),
            pl.BlockSpec((1, N, C3), lambda b, s: (b, 0, 0)),
        ],
        out_specs=[
            pl.BlockSpec((1, chunk, C3), lambda b, s: (b, s, 0)),
            pl.BlockSpec((1, nsample, chunk, C3), lambda b, s: (b, 0, s, 0)),
        ],
        compiler_params=pltpu.CompilerParams(
            dimension_semantics=("parallel", "parallel")),
        out_shape=[
            jax.ShapeDtypeStruct((B, S, C3), jnp.float32),
            jax.ShapeDtypeStruct((B, nsample, S, C3), jnp.float32),
        ],
    )(fidx, table)
    return fps_pts, grp


def _mlp_chain(h, wrefs):
    for (w_ref, b_ref, s_ref, t_ref) in wrefs:
        h = jnp.dot(h, w_ref[...], preferred_element_type=jnp.float32)
        h = (h + b_ref[...]) * s_ref[...] + t_ref[...]
        h = jnp.maximum(h, 0.0)
    return h


def _attn_body(nsample, nlayers, *refs):
    grp_ref, fps_ref = refs[0], refs[1]
    wrefs = [tuple(refs[2 + 4 * i: 6 + 4 * i]) for i in range(nlayers)]
    ap_ref, ah_ref = refs[2 + 4 * nlayers], refs[3 + 4 * nlayers]
    out_ref = refs[4 + 4 * nlayers]

    g4 = grp_ref[0]
    ns, chunk, cin = g4.shape
    g = g4.reshape(ns * chunk, cin)
    gxn = g[:, :3]
    h = _mlp_chain(g, wrefs)
    hc = _mlp_chain(fps_ref[0], wrefs)
    cout = h.shape[1]

    cterm = jnp.dot(hc, ah_ref[...], preferred_element_type=jnp.float32)
    gsum = jnp.dot(gxn, ap_ref[...], preferred_element_type=jnp.float32)
    gsum = gsum + jnp.dot(h, ah_ref[...], preferred_element_type=jnp.float32)
    pre = cterm[None, :, :] - gsum.reshape(ns, chunk, cout)
    e = jnp.where(pre >= 0, pre, _ALPHA * pre)
    m = jnp.max(e, axis=0)
    ex = jnp.exp(e - m[None, :, :])
    att = ex / jnp.sum(ex, axis=0)[None, :, :]
    out_ref[0] = jnp.sum(att * h.reshape(ns, chunk, cout), axis=0)


def _attn_call(grp, fps_pts, layers, a_p, a_h, chunk):
    B, nsample, S, Cin = grp.shape
    Cout = a_h.shape[1]
    nlayers = len(layers)
    grid = (B, S // chunk)
    in_specs = [
        pl.BlockSpec((1, nsample, chunk, Cin), lambda b, s: (b, 0, s, 0)),
        pl.BlockSpec((1, chunk, Cin), lambda b, s: (b, s, 0)),
    ]
    args = [grp, fps_pts]
    for (wt, bb, sg, bt) in layers:
        for arr in (wt, bb, sg, bt):
            in_specs.append(pl.BlockSpec(arr.shape, lambda b, s: (0, 0)))
            args.append(arr)
    for arr in (a_p, a_h):
        in_specs.append(pl.BlockSpec(arr.shape, lambda b, s: (0, 0)))
        args.append(arr)
    out = pl.pallas_call(
        functools.partial(_attn_body, nsample, nlayers),
        grid=grid,
        in_specs=in_specs,
        out_specs=pl.BlockSpec((1, chunk, Cout), lambda b, s: (b, s, 0)),
        compiler_params=pltpu.CompilerParams(
            dimension_semantics=("parallel", "parallel")),
        out_shape=jax.ShapeDtypeStruct((B, S, Cout), jnp.float32),
    )(*args)
    return out


def _fp_body(nlayers, has_p1, head, *refs):
    i = 0
    x1_ref = refs[i]; i += 1
    x2_ref = refs[i]; i += 1
    p2_ref = refs[i]; i += 1
    p1_ref = None
    if has_p1:
        p1_ref = refs[i]; i += 1
    wrefs = [tuple(refs[i + 4 * j: i + 4 * j + 4]) for j in range(nlayers)]
    i += 4 * nlayers
    hrefs = None
    if head:
        hrefs = refs[i:i + 8]
        i += 8
    out_ref = refs[i]

    src = x1_ref[0]
    dst = x2_ref[0]
    p2 = p2_ref[0]
    chunk = src.shape[0]
    n2 = dst.shape[0]
    sqr = -2.0 * jax.lax.dot_general(
        src, dst, (((1,), (1,)), ((), ())), preferred_element_type=jnp.float32)
    sqr = sqr + jnp.sum(src * src, 1, keepdims=True)
    sqr = sqr + jnp.sum(dst * dst, axis=1)[None, :]
    col = jax.lax.broadcasted_iota(jnp.int32, (chunk, n2), 1)

    d = sqr
    ws = []
    rows = []
    for _ in range(3):
        mj = jnp.min(d, 1, keepdims=True)
        ij = jnp.min(jnp.where(d == mj, col, n2), 1, keepdims=True)
        oh = (col == ij).astype(jnp.float32)
        rows.append(jnp.dot(oh, p2, preferred_element_type=jnp.float32, precision=jax.lax.Precision.HIGHEST))
        ws.append(1.0 / (mj + 1e-8))
        d = jnp.where(col == ij, jnp.float32(np.inf), d)
    wsum = (ws[0] + ws[1]) + ws[2]
    interp = (ws[0] / wsum) * rows[0] + (ws[1] / wsum) * rows[1] \
        + (ws[2] / wsum) * rows[2]
    if has_p1:
        h = jnp.concatenate([p1_ref[0], interp], 1)
    else:
        h = interp
    h = _mlp_chain(h, wrefs)
    if head:
        c1w, c1b, c1s, c1t, c2w, c2b = hrefs[0], hrefs[1], hrefs[2], hrefs[3], hrefs[4], hrefs[5]
        h = jnp.dot(h, c1w[...], preferred_element_type=jnp.float32)
        h = (h + c1b[...]) * c1s[...] + c1t[...]
        h = jnp.maximum(h, 0.0)
        logits = jnp.dot(h, c2w[...], preferred_element_type=jnp.float32) + c2b[...]
        m = jnp.max(logits, 1, keepdims=True)
        sh = logits - m
        h = sh - jnp.log(jnp.sum(jnp.exp(sh), 1, keepdims=True))
    out_ref[0] = h


def _fp_call(x1r, x2r, p2, p1, layers, head_ws, chunk, cout):
    B, n1, _ = x1r.shape
    n2 = x2r.shape[1]
    grid = (B, n1 // chunk)
    in_specs = [
        pl.BlockSpec((1, chunk, 3), lambda b, s: (b, s, 0)),
        pl.BlockSpec((1, n2, 3), lambda b, s: (b, 0, 0)),
        pl.BlockSpec((1, n2, p2.shape[2]), lambda b, s: (b, 0, 0)),
    ]
    args = [x1r, x2r, p2]
    if p1 is not None:
        in_specs.append(pl.BlockSpec((1, chunk, p1.shape[2]), lambda b, s: (b, s, 0)))
        args.append(p1)
    for (wt, bb, sg, bt) in layers:
        for arr in (wt, bb, sg, bt):
            in_specs.append(pl.BlockSpec(arr.shape, lambda b, s: (0, 0)))
            args.append(arr)
    if head_ws is not None:
        for arr in head_ws:
            in_specs.append(pl.BlockSpec(arr.shape, lambda b, s: (0, 0)))
            args.append(arr)
    out = pl.pallas_call(
        functools.partial(_fp_body, len(layers), p1 is not None, head_ws is not None),
        grid=grid,
        in_specs=in_specs,
        out_specs=pl.BlockSpec((1, chunk, cout), lambda b, s: (b, s, 0)),
        compiler_params=pltpu.CompilerParams(
            dimension_semantics=("parallel", "parallel")),
        out_shape=jax.ShapeDtypeStruct((B, n1, cout), jnp.float32),
    )(*args)
    return out


def _prep_layers(mlp_params):
    out = []
    for l in mlp_params:
        wt = jnp.transpose(l['W'])
        bb = l['b'][None, :]
        sg = (_BN_SCALE * l['gamma'])[None, :]
        bt = l['beta'][None, :]
        out.append((wt, bb, sg, bt))
    return out


def kernel(xyz, points, params):
    B, _, N = xyz.shape
    xyz_c = xyz
    feats_r = jnp.transpose(points, (0, 2, 1))

    sa_chunks = [256, 256, 64, 16]
    l_xyz_c = [xyz_c]
    l_xyz_r = [jnp.transpose(xyz_c, (0, 2, 1))]
    l_feats = [feats_r]
    for li, cfg in enumerate(_SA_CFGS):
        p = params['sa%d' % (li + 1)]
        table = jnp.concatenate([l_xyz_r[-1], l_feats[-1]], -1)
        fidx = _fps_call(l_xyz_c[-1], cfg['npoint'])
        fps_pts, grp = _group_call(
            table, fidx, cfg['radius'] ** 2, cfg['nsample'], sa_chunks[li])
        layers = _prep_layers(p['mlp'])
        a_p = p['a'][:3, :]
        a_h = p['a'][3:, :]
        feats = _attn_call(grp, fps_pts, layers, a_p, a_h, sa_chunks[li])
        new_xyz_r = fps_pts[:, :, :3]
        l_xyz_c.append(jnp.transpose(new_xyz_r, (0, 2, 1)))
        l_xyz_r.append(new_xyz_r)
        l_feats.append(feats)

    fp_chunks = [64, 256, 256, 512]
    h = _fp_call(l_xyz_r[3], l_xyz_r[4], l_feats[4], l_feats[3],
                 _prep_layers(params['fp4']['mlp']), None, fp_chunks[0], 256)
    h = _fp_call(l_xyz_r[2], l_xyz_r[3], h, l_feats[2],
                 _prep_layers(params['fp3']['mlp']), None, fp_chunks[1], 256)
    h = _fp_call(l_xyz_r[1], l_xyz_r[2], h, l_feats[1],
                 _prep_layers(params['fp2']['mlp']), None, fp_chunks[2], 128)
    c1 = params['head']['c1']
    c2 = params['head']['c2']
    head_ws = (jnp.transpose(c1['W']), c1['b'][None, :],
               (_BN_SCALE * c1['gamma'])[None, :], c1['beta'][None, :],
               jnp.transpose(c2['W']), c2['b'][None, :],
               jnp.zeros((1, 1), jnp.float32), jnp.zeros((1, 1), jnp.float32))
    out = _fp_call(l_xyz_r[0], l_xyz_r[1], h, None,
                   _prep_layers(params['fp1']['mlp']), head_ws, fp_chunks[3], 13)
    return out


# speedup vs baseline: 2.4582x; 1.7393x over previous
import functools

import jax
import jax.numpy as jnp
import numpy as np
from jax.experimental import pallas as pl
from jax.experimental.pallas import tpu as pltpu
from jax.experimental.pallas import tpu_sc as plsc

_SA_CFGS = [
    {'npoint': 1024, 'radius': 0.1, 'nsample': 32},
    {'npoint': 256, 'radius': 0.2, 'nsample': 32},
    {'npoint': 64, 'radius': 0.4, 'nsample': 32},
    {'npoint': 16, 'radius': 0.8, 'nsample': 32},
]
_ALPHA = 0.2
_BN_SCALE = 1.0 / np.sqrt(1.0 + 1e-5)


def _fps_body(npoint, N, xyz_ref, idx_ref):
    xs = xyz_ref[:, 0, :]
    ys = xyz_ref[:, 1, :]
    zs = xyz_ref[:, 2, :]
    B = xs.shape[0]
    iota = jax.lax.broadcasted_iota(jnp.int32, (B, N), 1)
    iota_np = jax.lax.broadcasted_iota(jnp.int32, (B, npoint), 1)

    idx_ref[:, 0, :] = jnp.zeros((B, npoint), jnp.int32)

    def body(i, st):
        dist, far = st
        idx_ref[:, 0, :] = jnp.where(iota_np == i, far, idx_ref[:, 0, :])
        oh = (iota == far).astype(jnp.float32)
        cx = jnp.sum(xs * oh, 1, keepdims=True)
        cy = jnp.sum(ys * oh, 1, keepdims=True)
        cz = jnp.sum(zs * oh, 1, keepdims=True)
        dx = xs - cx
        dy = ys - cy
        dz = zs - cz
        d = dx * dx + dy * dy + dz * dz
        dist = jnp.minimum(dist, d)
        m = jnp.max(dist, 1, keepdims=True)
        far = jnp.min(jnp.where(dist == m, iota, N), 1, keepdims=True)
        return dist, far

    dist0 = jnp.full((B, N), 1e10, jnp.float32)
    far0 = jnp.zeros((B, 1), jnp.int32)
    jax.lax.fori_loop(0, npoint, body, (dist0, far0))


def _fps_call(xyz_c, npoint):
    B, _, N = xyz_c.shape
    out = pl.pallas_call(
        functools.partial(_fps_body, npoint, N),
        out_shape=jax.ShapeDtypeStruct((B, 1, npoint), jnp.int32),
    )(xyz_c)
    return out


def _group_body(N, C3, nsample, r2, fidx_ref, table_ref, fps_ref, idx_out_ref):
    table = table_ref[0]
    fidx = fidx_ref[0, 0, :]
    chunk = fidx.shape[0]
    base = pl.program_id(0) * N
    col = jax.lax.broadcasted_iota(jnp.int32, (chunk, N), 1)

    foh = (col == fidx[:, None]).astype(jnp.float32)
    fps_pts = jnp.dot(foh, table, preferred_element_type=jnp.float32, precision=jax.lax.Precision.HIGHEST)
    fps_ref[0] = fps_pts
    src3 = fps_pts[:, :3]
    t3 = table[:, :3]
    sqr = -2.0 * jax.lax.dot_general(
        src3, t3, (((1,), (1,)), ((), ())), preferred_element_type=jnp.float32)
    sqr = sqr + jnp.sum(src3 * src3, 1, keepdims=True)
    sqr = sqr + jnp.sum(t3 * t3, axis=1)[None, :]
    avail = sqr <= r2

    idx0 = None
    for k in range(nsample):
        cand = jnp.where(avail, col, N)
        ik = jnp.min(cand, 1, keepdims=True)
        if k == 0:
            idx0 = jnp.minimum(ik, N - 1)
            iku = idx0
        else:
            iku = jnp.where(ik < N, ik, idx0)
        avail = jnp.logical_and(avail, col != ik)
        idx_out_ref[0, k] = iku[:, 0] + base


def _group_call(table, fidx, r2, nsample, chunk):
    B, N, C3 = table.shape
    S = fidx.shape[2]
    grid = (B, S // chunk)
    fps_pts, grp = pl.pallas_call(
        functools.partial(_group_body, N, C3, nsample, r2),
        grid=grid,
        in_specs=[
            pl.BlockSpec((1, 1, chunk), lambda b, s: (b, 0, s)),
            pl.BlockSpec((1, N, C3), lambda b, s: (b, 0, 0)),
        ],
        out_specs=[
            pl.BlockSpec((1, chunk, C3), lambda b, s: (b, s, 0)),
            pl.BlockSpec((1, nsample, chunk), lambda b, s: (b, 0, s)),
        ],
        compiler_params=pltpu.CompilerParams(
            dimension_semantics=("parallel", "parallel")),
        out_shape=[
            jax.ShapeDtypeStruct((B, S, C3), jnp.float32),
            jax.ShapeDtypeStruct((B, nsample, S), jnp.int32),
        ],
    )(fidx, table)
    return fps_pts, grp


def _sc_gather(table_flat, idx_flat):
    R = idx_flat.shape[0]
    Dp = table_flat.shape[1]
    info = plsc.get_sparse_core_info()
    nw = info.num_cores * info.num_subcores
    b_per_w = R // nw
    rps = min(b_per_w, max(8, 98304 // Dp))
    n_sub = b_per_w // rps
    mesh = plsc.VectorSubcoreMesh(core_axis_name="c", subcore_axis_name="s")

    @functools.partial(
        pl.kernel, mesh=mesh,
        out_type=jax.ShapeDtypeStruct((R, Dp), jnp.float32),
        scratch_types=[
            pltpu.VMEM((b_per_w,), jnp.int32),
            pltpu.VMEM((rps, Dp), jnp.float32),
            pltpu.SemaphoreType.DMA,
        ],
    )
    def k(table_hbm, idx_hbm, out_hbm, idx_v, rows_v, sem):
        wid = jax.lax.axis_index("s") * info.num_cores + jax.lax.axis_index("c")
        base = wid * b_per_w
        pltpu.sync_copy(idx_hbm.at[pl.ds(base, b_per_w)], idx_v)
        for j in range(n_sub):
            pltpu.async_copy(
                table_hbm.at[idx_v.at[pl.ds(j * rps, rps)]], rows_v, sem).wait()
            pltpu.sync_copy(rows_v, out_hbm.at[pl.ds(base + j * rps, rps)])

    return k(table_flat, idx_flat)


def _mlp_chain(h, wrefs):
    for (w_ref, b_ref, s_ref, t_ref) in wrefs:
        h = jnp.dot(h, w_ref[...], preferred_element_type=jnp.float32)
        h = (h + b_ref[...]) * s_ref[...] + t_ref[...]
        h = jnp.maximum(h, 0.0)
    return h


def _attn_body(nsample, nlayers, cin, *refs):
    grp_ref, fps_ref = refs[0], refs[1]
    wrefs = [tuple(refs[2 + 4 * i: 6 + 4 * i]) for i in range(nlayers)]
    ap_ref, ah_ref = refs[2 + 4 * nlayers], refs[3 + 4 * nlayers]
    out_ref = refs[4 + 4 * nlayers]

    g4 = grp_ref[0]
    ns, chunk, dp = g4.shape
    rows = g4.reshape(ns * chunk, dp)
    cpts = fps_ref[0]
    src3 = jnp.broadcast_to(cpts[None, :, :3], (ns, chunk, 3)).reshape(ns * chunk, 3)
    gxn = rows[:, :3] - src3
    g = jnp.concatenate([gxn, rows[:, 3:cin]], 1)
    h = _mlp_chain(g, wrefs)
    hc = _mlp_chain(cpts, wrefs)
    cout = h.shape[1]

    cterm = jnp.dot(hc, ah_ref[...], preferred_element_type=jnp.float32)
    gsum = jnp.dot(gxn, ap_ref[...], preferred_element_type=jnp.float32)
    gsum = gsum + jnp.dot(h, ah_ref[...], preferred_element_type=jnp.float32)
    pre = cterm[None, :, :] - gsum.reshape(ns, chunk, cout)
    e = jnp.where(pre >= 0, pre, _ALPHA * pre)
    m = jnp.max(e, axis=0)
    ex = jnp.exp(e - m[None, :, :])
    att = ex / jnp.sum(ex, axis=0)[None, :, :]
    out_ref[0] = jnp.sum(att * h.reshape(ns, chunk, cout), axis=0)


def _attn_call(grp, fps_pts, layers, a_p, a_h, chunk):
    B, nsample, S, Dp = grp.shape
    Cin = fps_pts.shape[2]
    Cout = a_h.shape[1]
    nlayers = len(layers)
    grid = (B, S // chunk)
    in_specs = [
        pl.BlockSpec((1, nsample, chunk, Dp), lambda b, s: (b, 0, s, 0)),
        pl.BlockSpec((1, chunk, Cin), lambda b, s: (b, s, 0)),
    ]
    args = [grp, fps_pts]
    for (wt, bb, sg, bt) in layers:
        for arr in (wt, bb, sg, bt):
            in_specs.append(pl.BlockSpec(arr.shape, lambda b, s: (0, 0)))
            args.append(arr)
    for arr in (a_p, a_h):
        in_specs.append(pl.BlockSpec(arr.shape, lambda b, s: (0, 0)))
        args.append(arr)
    out = pl.pallas_call(
        functools.partial(_attn_body, nsample, nlayers, Cin),
        grid=grid,
        in_specs=in_specs,
        out_specs=pl.BlockSpec((1, chunk, Cout), lambda b, s: (b, s, 0)),
        compiler_params=pltpu.CompilerParams(
            dimension_semantics=("parallel", "parallel")),
        out_shape=jax.ShapeDtypeStruct((B, S, Cout), jnp.float32),
    )(*args)
    return out


def _fp_body(nlayers, has_p1, head, *refs):
    i = 0
    x1_ref = refs[i]; i += 1
    x2_ref = refs[i]; i += 1
    p2_ref = refs[i]; i += 1
    p1_ref = None
    if has_p1:
        p1_ref = refs[i]; i += 1
    wrefs = [tuple(refs[i + 4 * j: i + 4 * j + 4]) for j in range(nlayers)]
    i += 4 * nlayers
    hrefs = None
    if head:
        hrefs = refs[i:i + 8]
        i += 8
    out_ref = refs[i]

    src = x1_ref[0]
    dst = x2_ref[0]
    p2 = p2_ref[0]
    chunk = src.shape[0]
    n2 = dst.shape[0]
    sqr = -2.0 * jax.lax.dot_general(
        src, dst, (((1,), (1,)), ((), ())), preferred_element_type=jnp.float32)
    sqr = sqr + jnp.sum(src * src, 1, keepdims=True)
    sqr = sqr + jnp.sum(dst * dst, axis=1)[None, :]
    col = jax.lax.broadcasted_iota(jnp.int32, (chunk, n2), 1)

    d = sqr
    ws = []
    rows = []
    for _ in range(3):
        mj = jnp.min(d, 1, keepdims=True)
        ij = jnp.min(jnp.where(d == mj, col, n2), 1, keepdims=True)
        oh = (col == ij).astype(jnp.float32)
        rows.append(jnp.dot(oh, p2, preferred_element_type=jnp.float32, precision=jax.lax.Precision.HIGHEST))
        ws.append(1.0 / (mj + 1e-8))
        d = jnp.where(col == ij, jnp.float32(np.inf), d)
    wsum = (ws[0] + ws[1]) + ws[2]
    interp = (ws[0] / wsum) * rows[0] + (ws[1] / wsum) * rows[1] \
        + (ws[2] / wsum) * rows[2]
    if has_p1:
        h = jnp.concatenate([p1_ref[0], interp], 1)
    else:
        h = interp
    h = _mlp_chain(h, wrefs)
    if head:
        c1w, c1b, c1s, c1t, c2w, c2b = hrefs[0], hrefs[1], hrefs[2], hrefs[3], hrefs[4], hrefs[5]
        h = jnp.dot(h, c1w[...], preferred_element_type=jnp.float32)
        h = (h + c1b[...]) * c1s[...] + c1t[...]
        h = jnp.maximum(h, 0.0)
        logits = jnp.dot(h, c2w[...], preferred_element_type=jnp.float32) + c2b[...]
        m = jnp.max(logits, 1, keepdims=True)
        sh = logits - m
        h = sh - jnp.log(jnp.sum(jnp.exp(sh), 1, keepdims=True))
    out_ref[0] = h


def _fp_call(x1r, x2r, p2, p1, layers, head_ws, chunk, cout):
    B, n1, _ = x1r.shape
    n2 = x2r.shape[1]
    grid = (B, n1 // chunk)
    in_specs = [
        pl.BlockSpec((1, chunk, 3), lambda b, s: (b, s, 0)),
        pl.BlockSpec((1, n2, 3), lambda b, s: (b, 0, 0)),
        pl.BlockSpec((1, n2, p2.shape[2]), lambda b, s: (b, 0, 0)),
    ]
    args = [x1r, x2r, p2]
    if p1 is not None:
        in_specs.append(pl.BlockSpec((1, chunk, p1.shape[2]), lambda b, s: (b, s, 0)))
        args.append(p1)
    for (wt, bb, sg, bt) in layers:
        for arr in (wt, bb, sg, bt):
            in_specs.append(pl.BlockSpec(arr.shape, lambda b, s: (0, 0)))
            args.append(arr)
    if head_ws is not None:
        for arr in head_ws:
            in_specs.append(pl.BlockSpec(arr.shape, lambda b, s: (0, 0)))
            args.append(arr)
    out = pl.pallas_call(
        functools.partial(_fp_body, len(layers), p1 is not None, head_ws is not None),
        grid=grid,
        in_specs=in_specs,
        out_specs=pl.BlockSpec((1, chunk, cout), lambda b, s: (b, s, 0)),
        compiler_params=pltpu.CompilerParams(
            dimension_semantics=("parallel", "parallel")),
        out_shape=jax.ShapeDtypeStruct((B, n1, cout), jnp.float32),
    )(*args)
    return out


def _prep_layers(mlp_params):
    out = []
    for l in mlp_params:
        wt = jnp.transpose(l['W'])
        bb = l['b'][None, :]
        sg = (_BN_SCALE * l['gamma'])[None, :]
        bt = l['beta'][None, :]
        out.append((wt, bb, sg, bt))
    return out


def kernel(xyz, points, params):
    B, _, N = xyz.shape
    xyz_c = xyz
    feats_r = jnp.transpose(points, (0, 2, 1))

    sa_chunks = [256, 256, 64, 16]
    l_xyz_c = [xyz_c]
    l_xyz_r = [jnp.transpose(xyz_c, (0, 2, 1))]
    l_feats = [feats_r]
    for li, cfg in enumerate(_SA_CFGS):
        p = params['sa%d' % (li + 1)]
        table = jnp.concatenate([l_xyz_r[-1], l_feats[-1]], -1)
        fidx = _fps_call(l_xyz_c[-1], cfg['npoint'])
        fps_pts, gidx = _group_call(
            table, fidx, cfg['radius'] ** 2, cfg['nsample'], sa_chunks[li])
        B_, N_, C3_ = table.shape
        dp = ((C3_ + 127) // 128) * 128
        tpad = jnp.pad(table, ((0, 0), (0, 0), (0, dp - C3_))).reshape(B_ * N_, dp)
        ns = cfg['nsample']
        S_ = gidx.shape[2]
        rows = _sc_gather(tpad, gidx.reshape(B_ * ns * S_))
        grp = rows.reshape(B_, ns, S_, dp)
        layers = _prep_layers(p['mlp'])
        a_p = p['a'][:3, :]
        a_h = p['a'][3:, :]
        feats = _attn_call(grp, fps_pts, layers, a_p, a_h, sa_chunks[li])
        new_xyz_r = fps_pts[:, :, :3]
        l_xyz_c.append(jnp.transpose(new_xyz_r, (0, 2, 1)))
        l_xyz_r.append(new_xyz_r)
        l_feats.append(feats)

    fp_chunks = [64, 256, 256, 512]
    h = _fp_call(l_xyz_r[3], l_xyz_r[4], l_feats[4], l_feats[3],
                 _prep_layers(params['fp4']['mlp']), None, fp_chunks[0], 256)
    h = _fp_call(l_xyz_r[2], l_xyz_r[3], h, l_feats[2],
                 _prep_layers(params['fp3']['mlp']), None, fp_chunks[1], 256)
    h = _fp_call(l_xyz_r[1], l_xyz_r[2], h, l_feats[1],
                 _prep_layers(params['fp2']['mlp']), None, fp_chunks[2], 128)
    c1 = params['head']['c1']
    c2 = params['head']['c2']
    head_ws = (jnp.transpose(c1['W']), c1['b'][None, :],
               (_BN_SCALE * c1['gamma'])[None, :], c1['beta'][None, :],
               jnp.transpose(c2['W']), c2['b'][None, :],
               jnp.zeros((1, 1), jnp.float32), jnp.zeros((1, 1), jnp.float32))
    out = _fp_call(l_xyz_r[0], l_xyz_r[1], h, None,
                   _prep_layers(params['fp1']['mlp']), head_ws, fp_chunks[3], 13)
    return out
